# baseline (speedup 1.0000x reference)
import functools

import jax
import jax.numpy as jnp
from jax import lax
from jax.experimental import pallas as pl
from jax.experimental.pallas import tpu as pltpu
from jax.experimental.pallas import tpu_sc as plsc

N = 10000
E = 320000
H = 128
L = 16
NC = 2
NS = 16
NW = NC * NS
EPW = E // NW
E2 = E // 2
EPWH = E2 // NW
CH = 40
NCH = EPWH // CH
NP = 10240
RPT = NP // NS
HL = H // L
HW = H // 2

@functools.cache
def _mesh():
    return plsc.VectorSubcoreMesh(
        core_axis_name="c", subcore_axis_name="s", num_cores=NC, num_subcores=NS
    )


GB = 3
CH2 = 88
NC2 = 57


def _chunk_off(ch):
    return pl.multiple_of(lax.min(ch * CH2, EPWH - CH2), 8)


def _gather_body(p_hbm, q_hbm, si_hbm, di_hbm, t_hbm, sidx, didx, bufp, bufq,
                 bufs, gsp, gsq, sss):
    c = lax.axis_index("c")
    s = lax.axis_index("s")
    wid = s * NC + c
    base = wid * EPWH
    pltpu.sync_copy(si_hbm.at[wid], sidx)
    pltpu.sync_copy(di_hbm.at[wid], didx)

    def issue(ch, b):
        pltpu.async_copy(p_hbm.at[sidx.at[ch]], bufp.at[b], gsp.at[b])
        pltpu.async_copy(q_hbm.at[didx.at[ch]], bufq.at[b], gsq.at[b])

    issue(0, 0)
    issue(1, 1)
    issue(2, 2)

    def chunk(ch, carry):
        b = lax.rem(ch, GB)
        pltpu.make_async_copy(p_hbm.at[sidx.at[ch]], bufp.at[b],
                              gsp.at[b]).wait()
        pltpu.make_async_copy(q_hbm.at[didx.at[ch]], bufq.at[b],
                              gsq.at[b]).wait()

        @pl.when(ch >= GB)
        def _():
            off2 = base + _chunk_off(ch - GB)
            pltpu.make_async_copy(bufs.at[b], t_hbm.at[pl.ds(off2, CH2)],
                                  sss.at[b]).wait()

        def row(i, carry2):
            for j in range(HL):
                sl = pl.ds(j * L, L)
                bufs[b, i, sl] = bufp[b, i, sl] + bufq[b, i, sl]
            return carry2

        lax.fori_loop(0, CH2, row, 0, unroll=2)
        off = base + _chunk_off(ch)
        pltpu.async_copy(bufs.at[b], t_hbm.at[pl.ds(off, CH2)], sss.at[b])

        @pl.when(ch + GB < NC2)
        def _():
            issue(ch + GB, b)

        return carry

    lax.fori_loop(0, NC2, chunk, 0)
    for k in range(NC2 - GB, NC2):
        b_ = k % GB
        off = base + _chunk_off(k)
        pltpu.make_async_copy(bufs.at[b_], t_hbm.at[pl.ds(off, CH2)],
                              sss.at[b_]).wait()


@functools.cache
def _gather_t():
    return pl.kernel(
        _gather_body,
        out_type=jax.ShapeDtypeStruct((E2, H), jnp.float32),
        mesh=_mesh(),
        scratch_types=[
            pltpu.VMEM((NC2, CH2), jnp.int32),
            pltpu.VMEM((NC2, CH2), jnp.int32),
            pltpu.VMEM((GB, CH2, H), jnp.float32),
            pltpu.VMEM((GB, CH2, H), jnp.float32),
            pltpu.VMEM((GB, CH2, H), jnp.float32),
            pltpu.SemaphoreType.DMA((GB,)),
            pltpu.SemaphoreType.DMA((GB,)),
            pltpu.SemaphoreType.DMA((GB,)),
        ],
    )


SB = 4
ZR = 16


def _scatter_body(e_hbm, di_hbm, out_hbm, ibuf, bufe, zbuf, shared, lsem,
                  csem, isem):
    c = lax.axis_index("c")
    s = lax.axis_index("s")
    wid = s * NC + c
    base = wid * EPWH

    def zrow(i, carry):
        for j in range(HL):
            zbuf[i, pl.ds(j * L, L)] = jnp.zeros((L,), jnp.float32)
        return carry

    lax.fori_loop(0, ZR, zrow, 0)
    my_off = pl.multiple_of(s * RPT, 8)
    for r in range(RPT // ZR):
        pltpu.sync_copy(zbuf, shared.at[pl.ds(my_off + r * ZR, ZR)])
    plsc.subcore_barrier()

    def lissue(ch, b):
        off = pl.multiple_of(base + ch * CH, 8)
        pltpu.async_copy(di_hbm.at[wid, ch], ibuf.at[b], isem.at[b])
        pltpu.async_copy(e_hbm.at[pl.ds(off, CH)], bufe.at[b], lsem.at[b])

    lissue(0, 0)
    lissue(1, 1)

    def chunk(ch, carry):
        b = lax.rem(ch, SB)
        off = pl.multiple_of(base + ch * CH, 8)
        pltpu.make_async_copy(di_hbm.at[wid, ch], ibuf.at[b],
                              isem.at[b]).wait()
        pltpu.make_async_copy(e_hbm.at[pl.ds(off, CH)], bufe.at[b],
                              lsem.at[b]).wait()

        nxt = ch + 2

        @pl.when(nxt < NCH)
        def _():
            b1 = lax.rem(nxt, SB)

            @pl.when(ch >= 2)
            def _():
                pltpu.make_async_copy(bufe.at[b1],
                                      shared.at[ibuf.at[b1]],
                                      csem.at[b1]).wait()

            lissue(nxt, b1)

        pltpu.async_copy(bufe.at[b], shared.at[ibuf.at[b]], csem.at[b],
                         add=True)
        return carry

    lax.fori_loop(0, NCH, chunk, 0)
    for k in range(NCH - SB, NCH):
        b_ = k % SB
        pltpu.make_async_copy(bufe.at[b_], shared.at[ibuf.at[b_]],
                              csem.at[b_]).wait()
    plsc.subcore_barrier()

    for r in range(RPT // ZR):
        pltpu.sync_copy(shared.at[pl.ds(my_off + r * ZR, ZR)], zbuf)
        pltpu.sync_copy(zbuf, out_hbm.at[c, pl.ds(my_off + r * ZR, ZR)])


@functools.cache
def _scatter_agg():
    return pl.kernel(
        _scatter_body,
        out_type=jax.ShapeDtypeStruct((NC, NP, H), jnp.float32),
        mesh=_mesh(),
        scratch_types=[
            pltpu.VMEM((SB, CH), jnp.int32),
            pltpu.VMEM((SB, CH, H), jnp.float32),
            pltpu.VMEM((ZR, H), jnp.float32),
            pltpu.VMEM_SHARED((NP, H), jnp.float32),
            pltpu.SemaphoreType.DMA((SB,)),
            pltpu.SemaphoreType.DMA((SB,)),
            pltpu.SemaphoreType.DMA((SB,)),
        ],
    )


_PREC = None


def _xform_body(x_ref, a_ref, b_ref, be1_ref, p_ref, q_ref):
    xb = x_ref[...]
    p_ref[...] = (
        jnp.dot(xb, a_ref[...], preferred_element_type=jnp.float32,
                precision=_PREC)
        + be1_ref[...]
    )
    q_ref[...] = jnp.dot(xb, b_ref[...], preferred_element_type=jnp.float32,
                         precision=_PREC)


def _tc_xform(x, a, b, be1s):
    g = 5
    rb = N // g
    return pl.pallas_call(
        _xform_body,
        grid=(g,),
        in_specs=[
            pl.BlockSpec((rb, H), lambda i: (i, 0)),
            pl.BlockSpec((H, H), lambda i: (0, 0)),
            pl.BlockSpec((H, H), lambda i: (0, 0)),
            pl.BlockSpec((1, H), lambda i: (0, 0)),
        ],
        out_specs=[pl.BlockSpec((rb, H), lambda i: (i, 0))] * 2,
        out_shape=[jax.ShapeDtypeStruct((N, H), jnp.float32)] * 2,
    )(x, a, b, be1s.reshape(1, H))


def _edge_body(t_ref, e_ref, c_ref, w2_ref, be2_ref, o_ref):
    eb = e_ref[...]
    h = jnp.maximum(
        t_ref[...]
        + jnp.dot(eb, c_ref[...], preferred_element_type=jnp.float32,
                  precision=_PREC),
        0.0,
    )
    o_ref[...] = (
        eb
        + jnp.dot(h, w2_ref[...], preferred_element_type=jnp.float32,
                  precision=_PREC)
        + be2_ref[...]
    )


def _edge_mlp(t, e, c, w2, be2s):
    g = 80
    rb = E2 // g
    return pl.pallas_call(
        _edge_body,
        grid=(g,),
        in_specs=[
            pl.BlockSpec((rb, H), lambda i: (i, 0)),
            pl.BlockSpec((rb, H), lambda i: (i, 0)),
            pl.BlockSpec((H, H), lambda i: (0, 0)),
            pl.BlockSpec((H, H), lambda i: (0, 0)),
            pl.BlockSpec((1, H), lambda i: (0, 0)),
        ],
        out_specs=pl.BlockSpec((rb, H), lambda i: (i, 0)),
        out_shape=jax.ShapeDtypeStruct((E2, H), jnp.float32),
    )(t, e, c, w2, be2s.reshape(1, H))


def _node_body(x_ref, pa_ref, pb_ref, w1_ref, bn1_ref, w2_ref, bn2_ref,
               a2_ref, b2_ref, b12_ref, o_ref, p_ref, q_ref):
    xb = x_ref[...]
    agg = pa_ref[0] + pa_ref[1] + pb_ref[0] + pb_ref[1]
    w1 = w1_ref[...]
    h = jnp.maximum(
        jnp.dot(xb, w1[:H], preferred_element_type=jnp.float32,
                precision=_PREC)
        + jnp.dot(agg, w1[H:], preferred_element_type=jnp.float32,
                  precision=_PREC)
        + bn1_ref[...],
        0.0,
    )
    xn = (
        xb
        + jnp.dot(h, w2_ref[...], preferred_element_type=jnp.float32,
                  precision=_PREC)
        + bn2_ref[...]
    )
    o_ref[...] = xn
    p_ref[...] = (
        jnp.dot(xn, a2_ref[...], preferred_element_type=jnp.float32,
                precision=_PREC)
        + b12_ref[...]
    )
    q_ref[...] = jnp.dot(xn, b2_ref[...], preferred_element_type=jnp.float32,
                         precision=_PREC)


def _node_mlp(x, parts_a, parts_b, w1, bn1s, w2, bn2s, a2, b2, b12s):
    g = 5
    rb = N // g
    return pl.pallas_call(
        _node_body,
        grid=(g,),
        in_specs=[
            pl.BlockSpec((rb, H), lambda i: (i, 0)),
            pl.BlockSpec((NC, rb, H), lambda i: (0, i, 0)),
            pl.BlockSpec((NC, rb, H), lambda i: (0, i, 0)),
            pl.BlockSpec((2 * H, H), lambda i: (0, 0)),
            pl.BlockSpec((1, H), lambda i: (0, 0)),
            pl.BlockSpec((H, H), lambda i: (0, 0)),
            pl.BlockSpec((1, H), lambda i: (0, 0)),
            pl.BlockSpec((H, H), lambda i: (0, 0)),
            pl.BlockSpec((H, H), lambda i: (0, 0)),
            pl.BlockSpec((1, H), lambda i: (0, 0)),
        ],
        out_specs=[pl.BlockSpec((rb, H), lambda i: (i, 0))] * 3,
        out_shape=[jax.ShapeDtypeStruct((N, H), jnp.float32)] * 3,
    )(x, parts_a, parts_b, w1, bn1s.reshape(1, H), w2, bn2s.reshape(1, H),
      a2, b2, b12s.reshape(1, H))


def _chunked_idx(a):
    return jnp.concatenate(
        [a[:, : (NC2 - 1) * CH2].reshape(NW, NC2 - 1, CH2),
         a[:, EPWH - CH2 :].reshape(NW, 1, CH2)], axis=1)


def kernel(x, edge_index, edge_attr, We1, be1, We2, be2, Wn1, bn1, Wn2, bn2):
    halves = []
    for hh in range(2):
        sl = slice(hh * E2, (hh + 1) * E2)
        sih = _chunked_idx(edge_index[0, sl].reshape(NW, EPWH))
        dih = _chunked_idx(edge_index[1, sl].reshape(NW, EPWH))
        di3 = edge_index[1, sl].reshape(NW, NCH, CH)
        halves.append((sih, dih, di3, edge_attr[sl]))

    (siA, diA, dA3, eA), (siB, diB, dB3, eB) = halves
    n_steps = We1.shape[0]
    p, q = _tc_xform(x, We1[0, :H], We1[0, H : 2 * H], be1[0])
    for s in range(n_steps):
        c = We1[s, 2 * H :]
        tA = _gather_t()(p, q, siA, diA)
        eA = _edge_mlp(tA, eA, c, We2[s], be2[s])
        tB = _gather_t()(p, q, siB, diB)
        eB = _edge_mlp(tB, eB, c, We2[s], be2[s])
        parts_a = _scatter_agg()(eA, dA3)
        parts_b = _scatter_agg()(eB, dB3)
        sn = min(s + 1, n_steps - 1)
        x, p, q = _node_mlp(x, parts_a, parts_b, Wn1[s], bn1[s], Wn2[s],
                            bn2[s], We1[sn, :H], We1[sn, H : 2 * H], be1[sn])
    return x, jnp.concatenate([eA, eB], axis=0)


# speedup vs baseline: 1.0716x; 1.0716x over previous
import functools

import jax
import jax.numpy as jnp
from jax import lax
from jax.experimental import pallas as pl
from jax.experimental.pallas import tpu as pltpu
from jax.experimental.pallas import tpu_sc as plsc

N = 10000
E = 320000
H = 128
L = 16
NC = 2
NS = 16
NW = NC * NS
EPW = E // NW
E2 = E // 2
EPWH = E2 // NW
CH = 80
NCM = 62
TAIL = EPWH - NCM * CH
NP = 10240
RPT = NP // NS
HL = H // L
HW = H // 2

@functools.cache
def _mesh():
    return plsc.VectorSubcoreMesh(
        core_axis_name="c", subcore_axis_name="s", num_cores=NC, num_subcores=NS
    )


GB = 2
CH2 = 96
NC2 = 53


def _chunk_off(ch):
    return pl.multiple_of(lax.min(ch * CH2, EPWH - CH2), 8)


def _gather_body(p_hbm, q_hbm, si_hbm, di_hbm, t_hbm, sidx, didx, bufp, bufq,
                 bufs, gsp, gsq, sss):
    c = lax.axis_index("c")
    s = lax.axis_index("s")
    wid = s * NC + c
    base = wid * EPWH
    pltpu.sync_copy(si_hbm.at[wid], sidx)
    pltpu.sync_copy(di_hbm.at[wid], didx)

    def issue(ch, b):
        pltpu.async_copy(p_hbm.at[sidx.at[ch]], bufp.at[b], gsp.at[b])
        pltpu.async_copy(q_hbm.at[didx.at[ch]], bufq.at[b], gsq.at[b])

    issue(0, 0)
    issue(1, 1)

    def chunk(ch, carry):
        b = lax.rem(ch, GB)
        pltpu.make_async_copy(p_hbm.at[sidx.at[ch]], bufp.at[b],
                              gsp.at[b]).wait()
        pltpu.make_async_copy(q_hbm.at[didx.at[ch]], bufq.at[b],
                              gsq.at[b]).wait()

        @pl.when(ch >= GB)
        def _():
            off2 = base + _chunk_off(ch - GB)
            pltpu.make_async_copy(bufs.at[b], t_hbm.at[pl.ds(off2, CH2)],
                                  sss.at[b]).wait()

        def row(i, carry2):
            for j in range(HL):
                sl = pl.ds(j * L, L)
                bufs[b, i, sl] = bufp[b, i, sl] + bufq[b, i, sl]
            return carry2

        lax.fori_loop(0, CH2, row, 0, unroll=2)
        off = base + _chunk_off(ch)
        pltpu.async_copy(bufs.at[b], t_hbm.at[pl.ds(off, CH2)], sss.at[b])

        @pl.when(ch + GB < NC2)
        def _():
            issue(ch + GB, b)

        return carry

    lax.fori_loop(0, NC2, chunk, 0)
    for k in range(NC2 - GB, NC2):
        b_ = k % GB
        off = base + _chunk_off(k)
        pltpu.make_async_copy(bufs.at[b_], t_hbm.at[pl.ds(off, CH2)],
                              sss.at[b_]).wait()


@functools.cache
def _gather_t():
    return pl.kernel(
        _gather_body,
        out_type=jax.ShapeDtypeStruct((E2, H), jnp.float32),
        mesh=_mesh(),
        scratch_types=[
            pltpu.VMEM((NC2, CH2), jnp.int32),
            pltpu.VMEM((NC2, CH2), jnp.int32),
            pltpu.VMEM((GB, CH2, H), jnp.float32),
            pltpu.VMEM((GB, CH2, H), jnp.float32),
            pltpu.VMEM((GB, CH2, H), jnp.float32),
            pltpu.SemaphoreType.DMA((GB,)),
            pltpu.SemaphoreType.DMA((GB,)),
            pltpu.SemaphoreType.DMA((GB,)),
        ],
    )


SB = 4
ZR = 16


def _scatter_body(e_hbm, di_hbm, dit_hbm, out_hbm, ibuf, itail, bufe, zbuf,
                  shared, lsem, csem, isem):
    c = lax.axis_index("c")
    s = lax.axis_index("s")
    wid = s * NC + c
    base = wid * EPWH

    def zrow(i, carry):
        for j in range(HL):
            zbuf[i, pl.ds(j * L, L)] = jnp.zeros((L,), jnp.float32)
        return carry

    lax.fori_loop(0, ZR, zrow, 0)
    my_off = pl.multiple_of(s * RPT, 8)
    for r in range(RPT // ZR):
        pltpu.sync_copy(zbuf, shared.at[pl.ds(my_off + r * ZR, ZR)])
    plsc.subcore_barrier()

    def lissue(ch, b):
        off = pl.multiple_of(base + ch * CH, 8)
        pltpu.async_copy(di_hbm.at[wid, ch], ibuf.at[b], isem.at[b])
        pltpu.async_copy(e_hbm.at[pl.ds(off, CH)], bufe.at[b], lsem.at[b])

    lissue(0, 0)
    lissue(1, 1)

    def chunk(ch, carry):
        b = lax.rem(ch, SB)
        off = pl.multiple_of(base + ch * CH, 8)
        pltpu.make_async_copy(di_hbm.at[wid, ch], ibuf.at[b],
                              isem.at[b]).wait()
        pltpu.make_async_copy(e_hbm.at[pl.ds(off, CH)], bufe.at[b],
                              lsem.at[b]).wait()

        nxt = ch + 2

        @pl.when(nxt < NCM)
        def _():
            b1 = lax.rem(nxt, SB)

            @pl.when(ch >= 2)
            def _():
                pltpu.make_async_copy(bufe.at[b1],
                                      shared.at[ibuf.at[b1]],
                                      csem.at[b1]).wait()

            lissue(nxt, b1)

        pltpu.async_copy(bufe.at[b], shared.at[ibuf.at[b]], csem.at[b],
                         add=True)
        return carry

    lax.fori_loop(0, NCM, chunk, 0)
    for k in range(NCM - SB, NCM):
        b_ = k % SB
        pltpu.make_async_copy(bufe.at[b_], shared.at[ibuf.at[b_]],
                              csem.at[b_]).wait()

    pltpu.sync_copy(dit_hbm.at[wid], itail)
    off_t = pl.multiple_of(base + NCM * CH, 8)
    pltpu.sync_copy(e_hbm.at[pl.ds(off_t, TAIL)], bufe.at[0, pl.ds(0, TAIL)])
    pltpu.sync_copy(bufe.at[0, pl.ds(0, TAIL)], shared.at[itail], add=True)
    plsc.subcore_barrier()

    for r in range(RPT // ZR):
        pltpu.sync_copy(shared.at[pl.ds(my_off + r * ZR, ZR)], zbuf)
        pltpu.sync_copy(zbuf, out_hbm.at[c, pl.ds(my_off + r * ZR, ZR)])


@functools.cache
def _scatter_agg():
    return pl.kernel(
        _scatter_body,
        out_type=jax.ShapeDtypeStruct((NC, NP, H), jnp.float32),
        mesh=_mesh(),
        scratch_types=[
            pltpu.VMEM((SB, CH), jnp.int32),
            pltpu.VMEM((TAIL,), jnp.int32),
            pltpu.VMEM((SB, CH, H), jnp.float32),
            pltpu.VMEM((ZR, H), jnp.float32),
            pltpu.VMEM_SHARED((NP, H), jnp.float32),
            pltpu.SemaphoreType.DMA((SB,)),
            pltpu.SemaphoreType.DMA((SB,)),
            pltpu.SemaphoreType.DMA((SB,)),
        ],
    )


_PREC = None


def _xform_body(x_ref, a_ref, b_ref, be1_ref, p_ref, q_ref):
    xb = x_ref[...]
    p_ref[...] = (
        jnp.dot(xb, a_ref[...], preferred_element_type=jnp.float32,
                precision=_PREC)
        + be1_ref[...]
    )
    q_ref[...] = jnp.dot(xb, b_ref[...], preferred_element_type=jnp.float32,
                         precision=_PREC)


def _tc_xform(x, a, b, be1s):
    g = 5
    rb = N // g
    return pl.pallas_call(
        _xform_body,
        grid=(g,),
        in_specs=[
            pl.BlockSpec((rb, H), lambda i: (i, 0)),
            pl.BlockSpec((H, H), lambda i: (0, 0)),
            pl.BlockSpec((H, H), lambda i: (0, 0)),
            pl.BlockSpec((1, H), lambda i: (0, 0)),
        ],
        out_specs=[pl.BlockSpec((rb, H), lambda i: (i, 0))] * 2,
        out_shape=[jax.ShapeDtypeStruct((N, H), jnp.float32)] * 2,
    )(x, a, b, be1s.reshape(1, H))


def _edge_body(t_ref, e_ref, c_ref, w2_ref, be2_ref, o_ref):
    eb = e_ref[...]
    h = jnp.maximum(
        t_ref[...]
        + jnp.dot(eb, c_ref[...], preferred_element_type=jnp.float32,
                  precision=_PREC),
        0.0,
    )
    o_ref[...] = (
        eb
        + jnp.dot(h, w2_ref[...], preferred_element_type=jnp.float32,
                  precision=_PREC)
        + be2_ref[...]
    )


def _edge_mlp(t, e, c, w2, be2s):
    g = 80
    rb = E2 // g
    return pl.pallas_call(
        _edge_body,
        grid=(g,),
        in_specs=[
            pl.BlockSpec((rb, H), lambda i: (i, 0)),
            pl.BlockSpec((rb, H), lambda i: (i, 0)),
            pl.BlockSpec((H, H), lambda i: (0, 0)),
            pl.BlockSpec((H, H), lambda i: (0, 0)),
            pl.BlockSpec((1, H), lambda i: (0, 0)),
        ],
        out_specs=pl.BlockSpec((rb, H), lambda i: (i, 0)),
        out_shape=jax.ShapeDtypeStruct((E2, H), jnp.float32),
    )(t, e, c, w2, be2s.reshape(1, H))


def _node_body(x_ref, pa_ref, pb_ref, w1_ref, bn1_ref, w2_ref, bn2_ref,
               a2_ref, b2_ref, b12_ref, o_ref, p_ref, q_ref):
    xb = x_ref[...]
    agg = pa_ref[0] + pa_ref[1] + pb_ref[0] + pb_ref[1]
    w1 = w1_ref[...]
    h = jnp.maximum(
        jnp.dot(xb, w1[:H], preferred_element_type=jnp.float32,
                precision=_PREC)
        + jnp.dot(agg, w1[H:], preferred_element_type=jnp.float32,
                  precision=_PREC)
        + bn1_ref[...],
        0.0,
    )
    xn = (
        xb
        + jnp.dot(h, w2_ref[...], preferred_element_type=jnp.float32,
                  precision=_PREC)
        + bn2_ref[...]
    )
    o_ref[...] = xn
    p_ref[...] = (
        jnp.dot(xn, a2_ref[...], preferred_element_type=jnp.float32,
                precision=_PREC)
        + b12_ref[...]
    )
    q_ref[...] = jnp.dot(xn, b2_ref[...], preferred_element_type=jnp.float32,
                         precision=_PREC)


def _node_mlp(x, parts_a, parts_b, w1, bn1s, w2, bn2s, a2, b2, b12s):
    g = 5
    rb = N // g
    return pl.pallas_call(
        _node_body,
        grid=(g,),
        in_specs=[
            pl.BlockSpec((rb, H), lambda i: (i, 0)),
            pl.BlockSpec((NC, rb, H), lambda i: (0, i, 0)),
            pl.BlockSpec((NC, rb, H), lambda i: (0, i, 0)),
            pl.BlockSpec((2 * H, H), lambda i: (0, 0)),
            pl.BlockSpec((1, H), lambda i: (0, 0)),
            pl.BlockSpec((H, H), lambda i: (0, 0)),
            pl.BlockSpec((1, H), lambda i: (0, 0)),
            pl.BlockSpec((H, H), lambda i: (0, 0)),
            pl.BlockSpec((H, H), lambda i: (0, 0)),
            pl.BlockSpec((1, H), lambda i: (0, 0)),
        ],
        out_specs=[pl.BlockSpec((rb, H), lambda i: (i, 0))] * 3,
        out_shape=[jax.ShapeDtypeStruct((N, H), jnp.float32)] * 3,
    )(x, parts_a, parts_b, w1, bn1s.reshape(1, H), w2, bn2s.reshape(1, H),
      a2, b2, b12s.reshape(1, H))


def _chunked_idx(a):
    return jnp.concatenate(
        [a[:, : (NC2 - 1) * CH2].reshape(NW, NC2 - 1, CH2),
         a[:, EPWH - CH2 :].reshape(NW, 1, CH2)], axis=1)


def kernel(x, edge_index, edge_attr, We1, be1, We2, be2, Wn1, bn1, Wn2, bn2):
    halves = []
    for hh in range(2):
        sl = slice(hh * E2, (hh + 1) * E2)
        sih = _chunked_idx(edge_index[0, sl].reshape(NW, EPWH))
        dih = _chunked_idx(edge_index[1, sl].reshape(NW, EPWH))
        dh = edge_index[1, sl].reshape(NW, EPWH)
        di3 = dh[:, : NCM * CH].reshape(NW, NCM, CH)
        dit = dh[:, NCM * CH :]
        halves.append((sih, dih, di3, dit, edge_attr[sl]))

    (siA, diA, dA3, dAt, eA), (siB, diB, dB3, dBt, eB) = halves
    n_steps = We1.shape[0]
    p, q = _tc_xform(x, We1[0, :H], We1[0, H : 2 * H], be1[0])
    for s in range(n_steps):
        c = We1[s, 2 * H :]
        tA = _gather_t()(p, q, siA, diA)
        eA = _edge_mlp(tA, eA, c, We2[s], be2[s])
        tB = _gather_t()(p, q, siB, diB)
        eB = _edge_mlp(tB, eB, c, We2[s], be2[s])
        parts_a = _scatter_agg()(eA, dA3, dAt)
        parts_b = _scatter_agg()(eB, dB3, dBt)
        sn = min(s + 1, n_steps - 1)
        x, p, q = _node_mlp(x, parts_a, parts_b, Wn1[s], bn1[s], Wn2[s],
                            bn2[s], We1[sn, :H], We1[sn, H : 2 * H], be1[sn])
    return x, jnp.concatenate([eA, eB], axis=0)
